# baseline (speedup 1.0000x reference)
import functools
import math

import jax
import jax.numpy as jnp
from jax import lax
from jax.experimental import pallas as pl
from jax.experimental.pallas import tpu as pltpu
from jax.experimental.pallas import tpu_sc as plsc

D_MODEL = 128
VOCAB = 100000
SCALE = math.sqrt(float(D_MODEL))

_NC = 2
_NS = 16
_NW = _NC * _NS

_B = 4096 * 200
_PER_W = _B // _NW
_CHUNK = 128
_NCHUNK = _PER_W // _CHUNK


def _scale_table(table):
    blk = 2000

    def body(t_ref, o_ref):
        o_ref[...] = t_ref[...] * SCALE

    return pl.pallas_call(
        body,
        out_shape=jax.ShapeDtypeStruct((VOCAB, D_MODEL), jnp.float32),
        grid=(VOCAB // blk,),
        in_specs=[pl.BlockSpec((blk, D_MODEL), lambda i: (i, 0))],
        out_specs=pl.BlockSpec((blk, D_MODEL), lambda i: (i, 0)),
    )(table)


_NBUF = 5


def _gather(idx, table):
    mesh = plsc.VectorSubcoreMesh(core_axis_name="c", subcore_axis_name="s")

    @functools.partial(
        pl.kernel,
        mesh=mesh,
        out_type=jax.ShapeDtypeStruct((_B, D_MODEL), jnp.float32),
        scratch_types=[
            pltpu.VMEM((_NCHUNK, _CHUNK), jnp.int32),
            pltpu.VMEM((_NBUF, _CHUNK, D_MODEL), jnp.float32),
            pltpu.SemaphoreType.DMA,
            pltpu.SemaphoreType.DMA,
        ],
    )
    def k(idx_hbm, table_hbm, out_hbm, idx_v, rows_v, gsem, ssem):
        wid = lax.axis_index("s") * _NC + lax.axis_index("c")
        base = wid * _PER_W
        pltpu.sync_copy(idx_hbm.at[wid], idx_v)

        def gather_start(t, b):
            pltpu.async_copy(table_hbm.at[idx_v.at[t]], rows_v.at[b], gsem)

        def gather_wait(t, b):
            pltpu.make_async_copy(
                table_hbm.at[idx_v.at[t]], rows_v.at[b], gsem).wait()

        def scatter_start(t, b):
            pltpu.async_copy(
                rows_v.at[b], out_hbm.at[pl.ds(base + t * _CHUNK, _CHUNK)], ssem)

        def scatter_wait(t, b):
            pltpu.make_async_copy(
                rows_v.at[b], out_hbm.at[pl.ds(base + t * _CHUNK, _CHUNK)],
                ssem).wait()

        for b in range(_NBUF - 1):
            gather_start(b, b)

        @pl.loop(0, _NCHUNK, step=_NBUF)
        def step(j0):
            for b in range(_NBUF):
                t = j0 + b
                bn = (b + _NBUF - 1) % _NBUF
                gather_wait(t, b)
                scatter_start(t, b)
                if b == 0:
                    @pl.when(j0 > 0)
                    def _():
                        scatter_wait(t - 1, bn)
                else:
                    scatter_wait(t - 1, bn)

                @pl.when(t + _NBUF - 1 < _NCHUNK)
                def _():
                    gather_start(t + _NBUF - 1, bn)

        scatter_wait(_NCHUNK - 1, (_NCHUNK - 1) % _NBUF)

    return k(idx, table)


def kernel(x, table):
    idx = x.reshape(_NW, _NCHUNK, _CHUNK).astype(jnp.int32)
    scaled = _scale_table(table)
    out = _gather(idx, scaled)
    return out.reshape(4096, 200, D_MODEL)


# speedup vs baseline: 1.1560x; 1.1560x over previous
import functools
import math

import jax
import jax.numpy as jnp
from jax import lax
from jax.experimental import pallas as pl
from jax.experimental.pallas import tpu as pltpu
from jax.experimental.pallas import tpu_sc as plsc

D_MODEL = 128
VOCAB = 100000
SCALE = math.sqrt(float(D_MODEL))

_NC = 2
_NS = 16
_NW = _NC * _NS

_B = 4096 * 200
_PER_W = _B // _NW
_CHUNK = 128
_NCHUNK = _PER_W // _CHUNK


_NBUF = 5


def _gather(idx, table):
    mesh = plsc.VectorSubcoreMesh(core_axis_name="c", subcore_axis_name="s")

    @functools.partial(
        pl.kernel,
        mesh=mesh,
        out_type=jax.ShapeDtypeStruct((_B, D_MODEL), jnp.float32),
        scratch_types=[
            pltpu.VMEM((_NCHUNK, _CHUNK), jnp.int32),
            pltpu.VMEM((_NBUF, _CHUNK, D_MODEL), jnp.float32),
            pltpu.SemaphoreType.DMA,
            pltpu.SemaphoreType.DMA,
        ],
    )
    def k(idx_hbm, table_hbm, out_hbm, idx_v, rows_v, gsem, ssem):
        wid = lax.axis_index("s") * _NC + lax.axis_index("c")
        base = wid * _PER_W
        pltpu.sync_copy(idx_hbm.at[wid], idx_v)

        def gather_start(t, b):
            pltpu.async_copy(table_hbm.at[idx_v.at[t]], rows_v.at[b], gsem)

        def gather_wait(t, b):
            pltpu.make_async_copy(
                table_hbm.at[idx_v.at[t]], rows_v.at[b], gsem).wait()

        def scatter_start(t, b):
            pltpu.async_copy(
                rows_v.at[b], out_hbm.at[pl.ds(base + t * _CHUNK, _CHUNK)], ssem)

        def scatter_wait(t, b):
            pltpu.make_async_copy(
                rows_v.at[b], out_hbm.at[pl.ds(base + t * _CHUNK, _CHUNK)],
                ssem).wait()

        def scale_slot(b):
            @plsc.parallel_loop(0, _CHUNK, unroll=4)
            def _(r):
                for c in range(D_MODEL // 16):
                    sl = pl.ds(c * 16, 16)
                    rows_v[b, r, sl] = rows_v[b, r, sl] * SCALE

        for b in range(_NBUF - 1):
            gather_start(b, b)

        @pl.loop(0, _NCHUNK, step=_NBUF)
        def step(j0):
            for b in range(_NBUF):
                t = j0 + b
                bn = (b + _NBUF - 1) % _NBUF
                gather_wait(t, b)
                scale_slot(b)
                scatter_start(t, b)
                if b == 0:
                    @pl.when(j0 > 0)
                    def _():
                        scatter_wait(t - 1, bn)
                else:
                    scatter_wait(t - 1, bn)

                @pl.when(t + _NBUF - 1 < _NCHUNK)
                def _():
                    gather_start(t + _NBUF - 1, bn)

        scatter_wait(_NCHUNK - 1, (_NCHUNK - 1) % _NBUF)

    return k(idx, table)


def kernel(x, table):
    idx = x.reshape(_NW, _NCHUNK, _CHUNK).astype(jnp.int32)
    out = _gather(idx, table)
    return out.reshape(4096, 200, D_MODEL)
